# baseline (speedup 1.0000x reference)
import functools
import math

import jax
import jax.numpy as jnp
from jax import lax
from jax.experimental import pallas as pl
from jax.experimental.pallas import tpu as pltpu
from jax.experimental.pallas import tpu_sc as plsc

_L = 16
_NW = 32
_CHUNK = 128

_PARAMS = pltpu.CompilerParams(use_tc_tiling_on_sc=True,
                               needs_layout_passes=False)


def _mesh():
    return plsc.VectorSubcoreMesh(core_axis_name="c", subcore_axis_name="s")


@functools.lru_cache(maxsize=None)
def _build_repad(vocab: int, dim: int):
    blk = 512

    def repad(tt_ref, out_ref):
        out_ref[:, :dim] = tt_ref[...].T
        out_ref[:, dim:] = jnp.zeros((blk, dim), jnp.float32)

    return pl.pallas_call(
        repad,
        grid=(pl.cdiv(vocab, blk),),
        in_specs=[pl.BlockSpec((dim, blk), lambda g: (0, g))],
        out_specs=pl.BlockSpec((blk, 2 * dim), lambda g: (g, 0)),
        out_shape=jax.ShapeDtypeStruct((vocab, 2 * dim), jnp.float32),
    )


@functools.lru_cache(maxsize=None)
def _build_lookup(seq: int, batch: int, vocab: int, dim: int):
    assert batch % _CHUNK == 0 and dim % _L == 0
    n_chunks = (seq * batch) // _CHUNK
    cpw = n_chunks // _NW
    cps = batch // _CHUNK
    scale = math.sqrt(dim)
    nk = dim // _L

    @functools.partial(
        pl.kernel,
        out_type=jax.ShapeDtypeStruct((n_chunks, _CHUNK, dim), jnp.float32),
        mesh=_mesh(),
        compiler_params=_PARAMS,
        scratch_types=[
            pltpu.VMEM((cpw, _CHUNK), jnp.int32),
            pltpu.VMEM((16, dim), jnp.float32),
            pltpu.VMEM((4, _CHUNK, 2 * dim), jnp.float32),
            pltpu.VMEM((2, _CHUNK, dim), jnp.float32),
            pltpu.SemaphoreType.DMA,
            pltpu.SemaphoreType.DMA,
            pltpu.SemaphoreType.DMA,
        ],
    )
    def lookup(x_hbm, tp_hbm, pe_hbm, out_hbm,
               idx_v, pe_v, gbuf, obuf, ssem, gsem, osem):
        wid = lax.axis_index("s") * 2 + lax.axis_index("c")
        base_c = pl.multiple_of(wid * cpw, 8)
        s0 = base_c // cps
        start8 = pl.multiple_of(
            lax.min((s0 // 8) * 8, jnp.int32(seq - 16)), 8)

        pltpu.make_async_copy(x_hbm.at[pl.ds(base_c, cpw)], idx_v,
                              ssem).start()
        pltpu.make_async_copy(pe_hbm.at[pl.ds(start8, 16)], pe_v,
                              ssem).start()
        pltpu.make_async_copy(x_hbm.at[pl.ds(base_c, cpw)], idx_v,
                              ssem).wait()
        pltpu.make_async_copy(pe_hbm.at[pl.ds(start8, 16)], pe_v,
                              ssem).wait()

        def gather(t, slot):
            return pltpu.make_async_copy(
                tp_hbm.at[idx_v.at[t]], gbuf.at[slot], gsem)

        def put(t, slot):
            return pltpu.make_async_copy(
                obuf.at[slot], out_hbm.at[base_c + t], osem)

        for b in range(4):
            gather(b, b).start()

        def step(t, carry):
            slot = lax.rem(t, 4)
            oslot = lax.rem(t, 2)
            gather(t, slot).wait()

            @pl.when(t >= 2)
            def _():
                put(t, oslot).wait()

            s_loc = (base_c + t) // cps - start8
            pe_regs = [pe_v[s_loc, pl.ds(k * _L, _L)] for k in range(nk)]
            g_ref = gbuf.at[slot]
            o_ref = obuf.at[oslot]

            def row(i, c2):
                for k in range(nk):
                    sl = pl.ds(k * _L, _L)
                    o_ref[i, sl] = g_ref[i, sl] * scale + pe_regs[k]
                return c2
            lax.fori_loop(0, _CHUNK, row, 0, unroll=2)

            put(t, oslot).start()

            @pl.when(t + 4 < cpw)
            def _():
                gather(t + 4, slot).start()
            return carry

        lax.fori_loop(0, cpw, step, 0)
        put(cpw - 2, lax.rem(cpw - 2, 2)).wait()
        put(cpw - 1, lax.rem(cpw - 1, 2)).wait()

    return lookup


def kernel(x, table, pe, pos):
    seq, batch = x.shape
    vocab, dim = table.shape
    tp = _build_repad(vocab, dim)(table.T)
    pe_rows = lax.dynamic_slice_in_dim(pe, pos, seq, axis=0).reshape(seq, dim)
    x2 = x.astype(jnp.int32).reshape((seq * batch) // _CHUNK, _CHUNK)
    out = _build_lookup(seq, batch, vocab, dim)(x2, tp, pe_rows)
    return out.reshape(seq, batch, dim)


# speedup vs baseline: 1.5267x; 1.5267x over previous
import functools
import math

import jax
import jax.numpy as jnp
from jax import lax
from jax.experimental import pallas as pl
from jax.experimental.pallas import tpu as pltpu
from jax.experimental.pallas import tpu_sc as plsc

_L = 16
_NW = 32
_CHUNK = 128

_PARAMS = pltpu.CompilerParams(use_tc_tiling_on_sc=True,
                               needs_layout_passes=False)


def _mesh():
    return plsc.VectorSubcoreMesh(core_axis_name="c", subcore_axis_name="s")


@functools.lru_cache(maxsize=None)
def _build_lookup(seq: int, batch: int, vocab: int, dim: int):
    assert batch % _CHUNK == 0 and dim % _L == 0
    n_chunks = (seq * batch) // _CHUNK
    cpw = n_chunks // _NW
    cps = batch // _CHUNK
    scale = math.sqrt(dim)
    nk = dim // _L

    @functools.partial(
        pl.kernel,
        out_type=jax.ShapeDtypeStruct((n_chunks, _CHUNK, dim), jnp.float32),
        mesh=_mesh(),
        compiler_params=_PARAMS,
        scratch_types=[
            pltpu.VMEM((cpw, _CHUNK), jnp.int32),
            pltpu.VMEM((16, dim), jnp.float32),
            pltpu.VMEM((4, _CHUNK, 2 * dim), jnp.float32),
            pltpu.VMEM((2, _CHUNK, dim), jnp.float32),
            pltpu.SemaphoreType.DMA,
            pltpu.SemaphoreType.DMA,
            pltpu.SemaphoreType.DMA,
        ],
    )
    def lookup(x_hbm, tp_hbm, pe_hbm, out_hbm,
               idx_v, pe_v, gbuf, obuf, ssem, gsem, osem):
        wid = lax.axis_index("s") * 2 + lax.axis_index("c")
        base_c = pl.multiple_of(wid * cpw, 8)
        s0 = base_c // cps
        start8 = pl.multiple_of(
            lax.min((s0 // 8) * 8, jnp.int32(seq - 16)), 8)

        pltpu.make_async_copy(x_hbm.at[pl.ds(base_c, cpw)], idx_v,
                              ssem).start()
        pltpu.make_async_copy(pe_hbm.at[pl.ds(start8, 16)], pe_v,
                              ssem).start()
        pltpu.make_async_copy(x_hbm.at[pl.ds(base_c, cpw)], idx_v,
                              ssem).wait()
        pltpu.make_async_copy(pe_hbm.at[pl.ds(start8, 16)], pe_v,
                              ssem).wait()

        def gather(t, slot):
            return pltpu.make_async_copy(
                tp_hbm.at[idx_v.at[t]], gbuf.at[slot], gsem)

        def put(t, slot):
            return pltpu.make_async_copy(
                obuf.at[slot], out_hbm.at[base_c + t], osem)

        for b in range(4):
            gather(b, b).start()

        def step(t, carry):
            slot = lax.rem(t, 4)
            oslot = lax.rem(t, 2)
            gather(t, slot).wait()

            @pl.when(t >= 2)
            def _():
                put(t, oslot).wait()

            s_loc = (base_c + t) // cps - start8
            pe_regs = [pe_v[s_loc, pl.ds(k * _L, _L)] for k in range(nk)]
            g_ref = gbuf.at[slot]
            o_ref = obuf.at[oslot]

            def row(i, c2):
                for k in range(nk):
                    sl = pl.ds(k * _L, _L)
                    o_ref[i, sl] = g_ref[i, sl] * scale + pe_regs[k]
                return c2
            lax.fori_loop(0, _CHUNK, row, 0, unroll=2)

            put(t, oslot).start()

            @pl.when(t + 4 < cpw)
            def _():
                gather(t + 4, slot).start()
            return carry

        lax.fori_loop(0, cpw, step, 0)
        put(cpw - 2, lax.rem(cpw - 2, 2)).wait()
        put(cpw - 1, lax.rem(cpw - 1, 2)).wait()

    return lookup


def kernel(x, table, pe, pos):
    seq, batch = x.shape
    vocab, dim = table.shape
    tp = jnp.pad(lax.optimization_barrier(table), ((0, 0), (0, dim)))
    pe_rows = lax.dynamic_slice_in_dim(pe, pos, seq, axis=0).reshape(seq, dim)
    x2 = x.astype(jnp.int32).reshape((seq * batch) // _CHUNK, _CHUNK)
    out = _build_lookup(seq, batch, vocab, dim)(x2, tp, pe_rows)
    return out.reshape(seq, batch, dim)
